# baseline (speedup 1.0000x reference)
import functools

import jax
import jax.numpy as jnp
from jax import lax
from jax.experimental import pallas as pl
from jax.experimental.pallas import tpu as pltpu
from jax.experimental.pallas import tpu_sc as plsc

T = 2048
D = 768
H = 768
E = 16
TOPK = 2
TK = T * TOPK
TM = 128
P = TK + E * TM
NB = P // TM

_NC = 2
_NS = 16
_NW = _NC * _NS


_TOK_PER_W = T // _NW


@functools.cache
def _make_sc_scatter_rows():
    @functools.partial(
        pl.kernel,
        name="sc_scatter_rows",
        out_type=jax.ShapeDtypeStruct((P, D), jnp.float32),
        mesh=plsc.VectorSubcoreMesh(core_axis_name="c", subcore_axis_name="s"),
        scratch_types=[
            pltpu.VMEM((_TOK_PER_W,), jnp.int32),
            pltpu.VMEM((_TOK_PER_W,), jnp.int32),
            pltpu.VMEM((_TOK_PER_W, D), jnp.float32),
            pltpu.SemaphoreType.DMA,
        ],
    )
    def _sc_scatter_rows(hf_hbm, pos0_hbm, pos1_hbm, out_hbm,
                         idx0_v, idx1_v, rows_v, sem):
        wid = lax.axis_index("s") * _NC + lax.axis_index("c")
        base = wid * _TOK_PER_W
        pltpu.sync_copy(pos0_hbm.at[pl.ds(base, _TOK_PER_W)], idx0_v)
        pltpu.sync_copy(pos1_hbm.at[pl.ds(base, _TOK_PER_W)], idx1_v)
        pltpu.sync_copy(hf_hbm.at[pl.ds(base, _TOK_PER_W)], rows_v)
        s0 = pltpu.async_copy(rows_v, out_hbm.at[idx0_v], sem)
        s1 = pltpu.async_copy(rows_v, out_hbm.at[idx1_v], sem)
        s0.wait()
        s1.wait()

    return _sc_scatter_rows


@functools.cache
def _make_sc_gather_outs():
    @functools.partial(
        pl.kernel,
        name="sc_gather_outs",
        out_type=(
            jax.ShapeDtypeStruct((T, D), jnp.float32),
            jax.ShapeDtypeStruct((T, D), jnp.float32),
        ),
        mesh=plsc.VectorSubcoreMesh(core_axis_name="c", subcore_axis_name="s"),
        scratch_types=[
            pltpu.VMEM((_TOK_PER_W,), jnp.int32),
            pltpu.VMEM((_TOK_PER_W,), jnp.int32),
            pltpu.VMEM((2, _TOK_PER_W, D), jnp.float32),
            pltpu.SemaphoreType.DMA,
            pltpu.SemaphoreType.DMA,
        ],
    )
    def _sc_gather_outs(outs_hbm, pos0_hbm, pos1_hbm, g0_hbm, g1_hbm,
                        idx0_v, idx1_v, rows_v, sg, sw):
        wid = lax.axis_index("s") * _NC + lax.axis_index("c")
        base = wid * _TOK_PER_W
        pltpu.sync_copy(pos0_hbm.at[pl.ds(base, _TOK_PER_W)], idx0_v)
        pltpu.sync_copy(pos1_hbm.at[pl.ds(base, _TOK_PER_W)], idx1_v)
        g0 = pltpu.async_copy(outs_hbm.at[idx0_v], rows_v.at[0], sg)
        g1 = pltpu.async_copy(outs_hbm.at[idx1_v], rows_v.at[1], sg)
        g0.wait()
        w0 = pltpu.async_copy(rows_v.at[0],
                              g0_hbm.at[pl.ds(base, _TOK_PER_W)], sw)
        g1.wait()
        w1 = pltpu.async_copy(rows_v.at[1],
                              g1_hbm.at[pl.ds(base, _TOK_PER_W)], sw)
        w0.wait()
        w1.wait()

    return _sc_gather_outs


def _grouped_body(be_ref, bv_ref, x_ref, w1_ref, w3_ref, w2_ref, o_ref):
    @pl.when(bv_ref[pl.program_id(0)] > 0)
    def _():
        x = x_ref[...].astype(jnp.bfloat16)
        a1 = lax.dot_general(x, w1_ref[0], (((1,), (1,)), ((), ())),
                             preferred_element_type=jnp.float32)
        a3 = lax.dot_general(x, w3_ref[0], (((1,), (1,)), ((), ())),
                             preferred_element_type=jnp.float32)
        act = (a1 * jax.nn.sigmoid(a1) * a3).astype(jnp.bfloat16)
        o_ref[...] = lax.dot_general(act, w2_ref[0], (((1,), (1,)), ((), ())),
                                     preferred_element_type=jnp.float32)


def _tc_grouped_ffn(block_ex, block_valid, x_sorted, W1, W3, W2):
    spec = pltpu.PrefetchScalarGridSpec(
        num_scalar_prefetch=2,
        grid=(NB,),
        in_specs=[
            pl.BlockSpec((TM, D), lambda b, be, bv: (b, 0)),
            pl.BlockSpec((1, H, D), lambda b, be, bv: (be[b], 0, 0)),
            pl.BlockSpec((1, H, D), lambda b, be, bv: (be[b], 0, 0)),
            pl.BlockSpec((1, D, H), lambda b, be, bv: (be[b], 0, 0)),
        ],
        out_specs=pl.BlockSpec((TM, D), lambda b, be, bv: (b, 0)),
    )
    return pl.pallas_call(
        _grouped_body,
        grid_spec=spec,
        out_shape=jax.ShapeDtypeStruct((P, D), jnp.float32),
        name="tc_grouped_ffn",
        compiler_params=pltpu.CompilerParams(
            dimension_semantics=("arbitrary",)),
    )(block_ex, block_valid, x_sorted, W1, W3, W2)


_TS = 256


def _shared_body(x_ref, w1_ref, w3_ref, w2_ref, g0_ref, g1_ref,
                 cw0_ref, cw1_ref, y_ref):
    x = x_ref[...]
    a1 = lax.dot_general(x, w1_ref[...], (((1,), (1,)), ((), ())),
                         preferred_element_type=jnp.float32)
    a3 = lax.dot_general(x, w3_ref[...], (((1,), (1,)), ((), ())),
                         preferred_element_type=jnp.float32)
    act = a1 * jax.nn.sigmoid(a1) * a3
    y = lax.dot_general(act, w2_ref[...], (((1,), (1,)), ((), ())),
                        preferred_element_type=jnp.float32)
    y_ref[...] = (y + cw0_ref[0, 0, :][:, None] * g0_ref[...]
                  + cw1_ref[0, 0, :][:, None] * g1_ref[...])


def _tc_shared_combine(hf, W1s, W3s, W2s, g0, g1, cw0, cw1):
    return pl.pallas_call(
        _shared_body,
        grid=(T // _TS,),
        in_specs=[
            pl.BlockSpec((_TS, D), lambda i: (i, 0)),
            pl.BlockSpec((H, D), lambda i: (0, 0)),
            pl.BlockSpec((H, D), lambda i: (0, 0)),
            pl.BlockSpec((D, H), lambda i: (0, 0)),
            pl.BlockSpec((_TS, D), lambda i: (i, 0)),
            pl.BlockSpec((_TS, D), lambda i: (i, 0)),
            pl.BlockSpec((1, 1, _TS), lambda i: (i, 0, 0)),
            pl.BlockSpec((1, 1, _TS), lambda i: (i, 0, 0)),
        ],
        out_specs=pl.BlockSpec((_TS, D), lambda i: (i, 0)),
        out_shape=jax.ShapeDtypeStruct((T, D), jnp.float32),
        name="tc_shared_combine",
    )(hf, W1s, W3s, W2s, g0, g1, cw0, cw1)


def _route_body(eb_ref, dst_ref, bex_ref, bval_ref):
    eb = eb_ref[...]
    lane = lax.broadcasted_iota(jnp.int32, (TK, E), 1)
    oh = (eb == lane).astype(jnp.int32)
    c = oh
    k = 1
    while k < TK:
        c = c + jnp.pad(c, ((k, 0), (0, 0)))[:TK]
        k *= 2
    counts = c[TK - 1:TK, :]
    padded = (((counts + TM - 1) // TM) * TM).astype(jnp.float32)
    le = (lax.broadcasted_iota(jnp.int32, (E, E), 0)
          <= lax.broadcasted_iota(jnp.int32, (E, E), 1)).astype(jnp.float32)
    padded_end = lax.dot_general(padded, le, (((1,), (0,)), ((), ())),
                                 preferred_element_type=jnp.float32)
    padded_start = (padded_end - padded).astype(jnp.int32)
    rank = jnp.sum(c * oh, axis=1, keepdims=True) - 1
    startj = jnp.sum(padded_start * oh, axis=1, keepdims=True)
    dst_ref[...] = jnp.broadcast_to(startj + rank, (TK, E))
    bstart = (lax.broadcasted_iota(jnp.int32, (NB, E), 0) * TM
              ).astype(jnp.float32)
    bex = jnp.sum((padded_end <= bstart).astype(jnp.int32),
                  axis=1, keepdims=True)
    bex_ref[...] = jnp.broadcast_to(jnp.minimum(bex, E - 1), (NB, E))
    real_lo = padded_end - padded
    real_hi = real_lo + counts.astype(jnp.float32)
    bval = jnp.sum(((bstart >= real_lo) & (bstart < real_hi)
                    ).astype(jnp.int32), axis=1, keepdims=True)
    bval_ref[...] = jnp.broadcast_to(bval, (NB, E))


def _tc_route_plan(e_bcast):
    return pl.pallas_call(
        _route_body,
        out_shape=(jax.ShapeDtypeStruct((TK, E), jnp.int32),
                   jax.ShapeDtypeStruct((NB, E), jnp.int32),
                   jax.ShapeDtypeStruct((NB, E), jnp.int32)),
        name="tc_route_plan",
    )(e_bcast)


def _route_metadata(ids):
    flat_e = ids.reshape(TK)
    e_bcast = jnp.broadcast_to(flat_e[:, None], (TK, E))
    dst_b, bex_b, bval_b = _tc_route_plan(e_bcast)
    pos = dst_b[:, 0].reshape(T, TOPK)
    return pos, bex_b[:, 0], bval_b[:, 0]


def kernel(h, Wg, W1, W2, W3, W1s, W2s, W3s):
    hf = h.reshape(T, D)
    scores = jax.nn.softmax(hf @ Wg.T, axis=-1)
    v1 = jnp.max(scores, axis=-1)
    a1 = jnp.argmax(scores, axis=-1).astype(jnp.int32)
    masked = jnp.where(
        jax.nn.one_hot(a1, E, dtype=jnp.bool_), -jnp.inf, scores)
    v2 = jnp.max(masked, axis=-1)
    a2 = jnp.argmax(masked, axis=-1).astype(jnp.int32)
    ids = jnp.stack([a1, a2], axis=1)
    vals = jnp.stack([v1, v2], axis=1)
    w = vals / jnp.sum(vals, axis=-1, keepdims=True)

    pos, block_ex, block_valid = _route_metadata(ids)
    pos0 = pos[:, 0].copy()
    pos1 = pos[:, 1].copy()
    cw0 = w[:, 0].reshape(T // _TS, 1, _TS)
    cw1 = w[:, 1].reshape(T // _TS, 1, _TS)

    x_sorted = _make_sc_scatter_rows()(hf, pos0, pos1)
    out_sorted = _tc_grouped_ffn(block_ex, block_valid, x_sorted,
                                 W1.astype(jnp.bfloat16),
                                 W3.astype(jnp.bfloat16),
                                 W2.astype(jnp.bfloat16))
    g0, g1 = _make_sc_gather_outs()(out_sorted, pos0, pos1)
    y = _tc_shared_combine(hf, W1s, W3s, W2s, g0, g1, cw0, cw1)
    return y.reshape(h.shape)


# speedup vs baseline: 1.2029x; 1.2029x over previous
import functools

import jax
import jax.numpy as jnp
from jax import lax
from jax.experimental import pallas as pl
from jax.experimental.pallas import tpu as pltpu
from jax.experimental.pallas import tpu_sc as plsc

T = 2048
D = 768
H = 768
E = 16
TOPK = 2
TK = T * TOPK
TM = 128
P = TK + E * TM
NB = P // TM

_NC = 2
_NS = 16
_NW = _NC * _NS


_TOK_PER_W = T // _NW


@functools.cache
def _make_sc_scatter_rows():
    @functools.partial(
        pl.kernel,
        name="sc_scatter_rows",
        out_type=jax.ShapeDtypeStruct((P, D), jnp.float32),
        mesh=plsc.VectorSubcoreMesh(core_axis_name="c", subcore_axis_name="s"),
        scratch_types=[
            pltpu.VMEM((_TOK_PER_W,), jnp.int32),
            pltpu.VMEM((_TOK_PER_W,), jnp.int32),
            pltpu.VMEM((_TOK_PER_W, D), jnp.float32),
            pltpu.SemaphoreType.DMA,
        ],
    )
    def _sc_scatter_rows(hf_hbm, pos0_hbm, pos1_hbm, out_hbm,
                         idx0_v, idx1_v, rows_v, sem):
        wid = lax.axis_index("s") * _NC + lax.axis_index("c")
        base = wid * _TOK_PER_W
        pltpu.sync_copy(pos0_hbm.at[pl.ds(base, _TOK_PER_W)], idx0_v)
        pltpu.sync_copy(pos1_hbm.at[pl.ds(base, _TOK_PER_W)], idx1_v)
        pltpu.sync_copy(hf_hbm.at[pl.ds(base, _TOK_PER_W)], rows_v)
        s0 = pltpu.async_copy(rows_v, out_hbm.at[idx0_v], sem)
        s1 = pltpu.async_copy(rows_v, out_hbm.at[idx1_v], sem)
        s0.wait()
        s1.wait()

    return _sc_scatter_rows


@functools.cache
def _make_sc_gather_outs():
    @functools.partial(
        pl.kernel,
        name="sc_gather_outs",
        out_type=(
            jax.ShapeDtypeStruct((T, D), jnp.float32),
            jax.ShapeDtypeStruct((T, D), jnp.float32),
        ),
        mesh=plsc.VectorSubcoreMesh(core_axis_name="c", subcore_axis_name="s"),
        scratch_types=[
            pltpu.VMEM((_TOK_PER_W,), jnp.int32),
            pltpu.VMEM((_TOK_PER_W,), jnp.int32),
            pltpu.VMEM((2, _TOK_PER_W, D), jnp.float32),
            pltpu.SemaphoreType.DMA,
            pltpu.SemaphoreType.DMA,
        ],
    )
    def _sc_gather_outs(outs_hbm, pos0_hbm, pos1_hbm, g0_hbm, g1_hbm,
                        idx0_v, idx1_v, rows_v, sg, sw):
        wid = lax.axis_index("s") * _NC + lax.axis_index("c")
        base = wid * _TOK_PER_W
        pltpu.sync_copy(pos0_hbm.at[pl.ds(base, _TOK_PER_W)], idx0_v)
        pltpu.sync_copy(pos1_hbm.at[pl.ds(base, _TOK_PER_W)], idx1_v)
        g0 = pltpu.async_copy(outs_hbm.at[idx0_v], rows_v.at[0], sg)
        g1 = pltpu.async_copy(outs_hbm.at[idx1_v], rows_v.at[1], sg)
        g0.wait()
        w0 = pltpu.async_copy(rows_v.at[0],
                              g0_hbm.at[pl.ds(base, _TOK_PER_W)], sw)
        g1.wait()
        w1 = pltpu.async_copy(rows_v.at[1],
                              g1_hbm.at[pl.ds(base, _TOK_PER_W)], sw)
        w0.wait()
        w1.wait()

    return _sc_gather_outs


def _grouped_body(be_ref, bv_ref, x_ref, w1_ref, w3_ref, w2_ref, o_ref):
    @pl.when(bv_ref[pl.program_id(0)] > 0)
    def _():
        x = x_ref[...]
        a1 = lax.dot_general(x, w1_ref[0], (((1,), (1,)), ((), ())),
                             preferred_element_type=jnp.float32)
        a3 = lax.dot_general(x, w3_ref[0], (((1,), (1,)), ((), ())),
                             preferred_element_type=jnp.float32)
        act = a1 * jax.nn.sigmoid(a1) * a3
        o_ref[...] = lax.dot_general(act, w2_ref[0], (((1,), (1,)), ((), ())),
                                     preferred_element_type=jnp.float32)


def _tc_grouped_ffn(block_ex, block_valid, x_sorted, W1, W3, W2):
    spec = pltpu.PrefetchScalarGridSpec(
        num_scalar_prefetch=2,
        grid=(NB,),
        in_specs=[
            pl.BlockSpec((TM, D), lambda b, be, bv: (b, 0)),
            pl.BlockSpec((1, H, D), lambda b, be, bv: (be[b], 0, 0)),
            pl.BlockSpec((1, H, D), lambda b, be, bv: (be[b], 0, 0)),
            pl.BlockSpec((1, D, H), lambda b, be, bv: (be[b], 0, 0)),
        ],
        out_specs=pl.BlockSpec((TM, D), lambda b, be, bv: (b, 0)),
    )
    return pl.pallas_call(
        _grouped_body,
        grid_spec=spec,
        out_shape=jax.ShapeDtypeStruct((P, D), jnp.float32),
        name="tc_grouped_ffn",
        compiler_params=pltpu.CompilerParams(
            dimension_semantics=("arbitrary",)),
    )(block_ex, block_valid, x_sorted, W1, W3, W2)


_TS = 256


def _shared_body(x_ref, w1_ref, w3_ref, w2_ref, g0_ref, g1_ref,
                 cw0_ref, cw1_ref, y_ref):
    x = x_ref[...]
    a1 = lax.dot_general(x, w1_ref[...], (((1,), (1,)), ((), ())),
                         preferred_element_type=jnp.float32)
    a3 = lax.dot_general(x, w3_ref[...], (((1,), (1,)), ((), ())),
                         preferred_element_type=jnp.float32)
    act = a1 * jax.nn.sigmoid(a1) * a3
    y = lax.dot_general(act, w2_ref[...], (((1,), (1,)), ((), ())),
                        preferred_element_type=jnp.float32)
    y_ref[...] = (y + cw0_ref[0, 0, :][:, None] * g0_ref[...]
                  + cw1_ref[0, 0, :][:, None] * g1_ref[...])


def _tc_shared_combine(hf, W1s, W3s, W2s, g0, g1, cw0, cw1):
    return pl.pallas_call(
        _shared_body,
        grid=(T // _TS,),
        in_specs=[
            pl.BlockSpec((_TS, D), lambda i: (i, 0)),
            pl.BlockSpec((H, D), lambda i: (0, 0)),
            pl.BlockSpec((H, D), lambda i: (0, 0)),
            pl.BlockSpec((D, H), lambda i: (0, 0)),
            pl.BlockSpec((_TS, D), lambda i: (i, 0)),
            pl.BlockSpec((_TS, D), lambda i: (i, 0)),
            pl.BlockSpec((1, 1, _TS), lambda i: (i, 0, 0)),
            pl.BlockSpec((1, 1, _TS), lambda i: (i, 0, 0)),
        ],
        out_specs=pl.BlockSpec((_TS, D), lambda i: (i, 0)),
        out_shape=jax.ShapeDtypeStruct((T, D), jnp.float32),
        name="tc_shared_combine",
    )(hf, W1s, W3s, W2s, g0, g1, cw0, cw1)


def _route_body(eb_ref, dst_ref, bex_ref, bval_ref):
    eb = eb_ref[...]
    lane = lax.broadcasted_iota(jnp.int32, (TK, E), 1)
    oh = (eb == lane).astype(jnp.int32)
    c = oh
    k = 1
    while k < TK:
        c = c + jnp.pad(c, ((k, 0), (0, 0)))[:TK]
        k *= 2
    counts = c[TK - 1:TK, :]
    padded = (((counts + TM - 1) // TM) * TM).astype(jnp.float32)
    le = (lax.broadcasted_iota(jnp.int32, (E, E), 0)
          <= lax.broadcasted_iota(jnp.int32, (E, E), 1)).astype(jnp.float32)
    padded_end = lax.dot_general(padded, le, (((1,), (0,)), ((), ())),
                                 preferred_element_type=jnp.float32)
    padded_start = (padded_end - padded).astype(jnp.int32)
    rank = jnp.sum(c * oh, axis=1, keepdims=True) - 1
    startj = jnp.sum(padded_start * oh, axis=1, keepdims=True)
    dst_ref[...] = jnp.broadcast_to(startj + rank, (TK, E))
    bstart = (lax.broadcasted_iota(jnp.int32, (NB, E), 0) * TM
              ).astype(jnp.float32)
    bex = jnp.sum((padded_end <= bstart).astype(jnp.int32),
                  axis=1, keepdims=True)
    bex_ref[...] = jnp.broadcast_to(jnp.minimum(bex, E - 1), (NB, E))
    real_lo = padded_end - padded
    real_hi = real_lo + counts.astype(jnp.float32)
    bval = jnp.sum(((bstart >= real_lo) & (bstart < real_hi)
                    ).astype(jnp.int32), axis=1, keepdims=True)
    bval_ref[...] = jnp.broadcast_to(bval, (NB, E))


def _tc_route_plan(e_bcast):
    return pl.pallas_call(
        _route_body,
        out_shape=(jax.ShapeDtypeStruct((TK, E), jnp.int32),
                   jax.ShapeDtypeStruct((NB, E), jnp.int32),
                   jax.ShapeDtypeStruct((NB, E), jnp.int32)),
        name="tc_route_plan",
    )(e_bcast)


def _route_metadata(ids):
    flat_e = ids.reshape(TK)
    e_bcast = jnp.broadcast_to(flat_e[:, None], (TK, E))
    dst_b, bex_b, bval_b = _tc_route_plan(e_bcast)
    pos = dst_b[:, 0].reshape(T, TOPK)
    return pos, bex_b[:, 0], bval_b[:, 0]


def kernel(h, Wg, W1, W2, W3, W1s, W2s, W3s):
    hf = h.reshape(T, D)
    scores = jax.nn.softmax(hf @ Wg.T, axis=-1)
    v1 = jnp.max(scores, axis=-1)
    a1 = jnp.argmax(scores, axis=-1).astype(jnp.int32)
    masked = jnp.where(
        jax.nn.one_hot(a1, E, dtype=jnp.bool_), -jnp.inf, scores)
    v2 = jnp.max(masked, axis=-1)
    a2 = jnp.argmax(masked, axis=-1).astype(jnp.int32)
    ids = jnp.stack([a1, a2], axis=1)
    vals = jnp.stack([v1, v2], axis=1)
    w = vals / jnp.sum(vals, axis=-1, keepdims=True)

    pos, block_ex, block_valid = _route_metadata(ids)
    pos0 = pos[:, 0].copy()
    pos1 = pos[:, 1].copy()
    cw0 = w[:, 0].reshape(T // _TS, 1, _TS)
    cw1 = w[:, 1].reshape(T // _TS, 1, _TS)

    x_sorted = _make_sc_scatter_rows()(hf, pos0, pos1)
    out_sorted = _tc_grouped_ffn(block_ex, block_valid, x_sorted, W1, W3, W2)
    g0, g1 = _make_sc_gather_outs()(out_sorted, pos0, pos1)
    y = _tc_shared_combine(hf, W1s, W3s, W2s, g0, g1, cw0, cw1)
    return y.reshape(h.shape)


# speedup vs baseline: 1.4268x; 1.1861x over previous
import functools

import jax
import jax.numpy as jnp
from jax import lax
from jax.experimental import pallas as pl
from jax.experimental.pallas import tpu as pltpu
from jax.experimental.pallas import tpu_sc as plsc

T = 2048
D = 768
H = 768
E = 16
TOPK = 2
TK = T * TOPK
TM = 256
P = TK + E * TM
NB = P // TM

_NC = 2
_NS = 16
_NW = _NC * _NS


_TOK_PER_W = T // _NW


@functools.cache
def _make_sc_scatter_rows():
    @functools.partial(
        pl.kernel,
        name="sc_scatter_rows",
        out_type=jax.ShapeDtypeStruct((P, D), jnp.float32),
        mesh=plsc.VectorSubcoreMesh(core_axis_name="c", subcore_axis_name="s"),
        scratch_types=[
            pltpu.VMEM((_TOK_PER_W,), jnp.int32),
            pltpu.VMEM((_TOK_PER_W,), jnp.int32),
            pltpu.VMEM((_TOK_PER_W, D), jnp.float32),
            pltpu.SemaphoreType.DMA,
        ],
    )
    def _sc_scatter_rows(hf_hbm, pos0_hbm, pos1_hbm, out_hbm,
                         idx0_v, idx1_v, rows_v, sem):
        wid = lax.axis_index("s") * _NC + lax.axis_index("c")
        base = wid * _TOK_PER_W
        pltpu.sync_copy(pos0_hbm.at[pl.ds(base, _TOK_PER_W)], idx0_v)
        pltpu.sync_copy(pos1_hbm.at[pl.ds(base, _TOK_PER_W)], idx1_v)
        pltpu.sync_copy(hf_hbm.at[pl.ds(base, _TOK_PER_W)], rows_v)
        s0 = pltpu.async_copy(rows_v, out_hbm.at[idx0_v], sem)
        s1 = pltpu.async_copy(rows_v, out_hbm.at[idx1_v], sem)
        s0.wait()
        s1.wait()

    return _sc_scatter_rows


@functools.cache
def _make_sc_gather_outs():
    @functools.partial(
        pl.kernel,
        name="sc_gather_outs",
        out_type=(
            jax.ShapeDtypeStruct((T, D), jnp.float32),
            jax.ShapeDtypeStruct((T, D), jnp.float32),
        ),
        mesh=plsc.VectorSubcoreMesh(core_axis_name="c", subcore_axis_name="s"),
        scratch_types=[
            pltpu.VMEM((_TOK_PER_W,), jnp.int32),
            pltpu.VMEM((_TOK_PER_W,), jnp.int32),
            pltpu.VMEM((2, _TOK_PER_W, D), jnp.float32),
            pltpu.SemaphoreType.DMA,
            pltpu.SemaphoreType.DMA,
        ],
    )
    def _sc_gather_outs(outs_hbm, pos0_hbm, pos1_hbm, g0_hbm, g1_hbm,
                        idx0_v, idx1_v, rows_v, sg, sw):
        wid = lax.axis_index("s") * _NC + lax.axis_index("c")
        base = wid * _TOK_PER_W
        pltpu.sync_copy(pos0_hbm.at[pl.ds(base, _TOK_PER_W)], idx0_v)
        pltpu.sync_copy(pos1_hbm.at[pl.ds(base, _TOK_PER_W)], idx1_v)
        g0 = pltpu.async_copy(outs_hbm.at[idx0_v], rows_v.at[0], sg)
        g1 = pltpu.async_copy(outs_hbm.at[idx1_v], rows_v.at[1], sg)
        g0.wait()
        w0 = pltpu.async_copy(rows_v.at[0],
                              g0_hbm.at[pl.ds(base, _TOK_PER_W)], sw)
        g1.wait()
        w1 = pltpu.async_copy(rows_v.at[1],
                              g1_hbm.at[pl.ds(base, _TOK_PER_W)], sw)
        w0.wait()
        w1.wait()

    return _sc_gather_outs


def _grouped_body(be_ref, bv_ref, x_ref, w1_ref, w3_ref, w2_ref, o_ref):
    @pl.when(bv_ref[pl.program_id(0)] > 0)
    def _():
        x = x_ref[...]
        a1 = lax.dot_general(x, w1_ref[0], (((1,), (1,)), ((), ())),
                             preferred_element_type=jnp.float32)
        a3 = lax.dot_general(x, w3_ref[0], (((1,), (1,)), ((), ())),
                             preferred_element_type=jnp.float32)
        act = a1 * jax.nn.sigmoid(a1) * a3
        o_ref[...] = lax.dot_general(act, w2_ref[0], (((1,), (1,)), ((), ())),
                                     preferred_element_type=jnp.float32)


def _tc_grouped_ffn(block_ex, block_valid, x_sorted, W1, W3, W2):
    spec = pltpu.PrefetchScalarGridSpec(
        num_scalar_prefetch=2,
        grid=(NB,),
        in_specs=[
            pl.BlockSpec((TM, D), lambda b, be, bv: (b, 0)),
            pl.BlockSpec((1, H, D), lambda b, be, bv: (be[b], 0, 0)),
            pl.BlockSpec((1, H, D), lambda b, be, bv: (be[b], 0, 0)),
            pl.BlockSpec((1, D, H), lambda b, be, bv: (be[b], 0, 0)),
        ],
        out_specs=pl.BlockSpec((TM, D), lambda b, be, bv: (b, 0)),
    )
    return pl.pallas_call(
        _grouped_body,
        grid_spec=spec,
        out_shape=jax.ShapeDtypeStruct((P, D), jnp.float32),
        name="tc_grouped_ffn",
        compiler_params=pltpu.CompilerParams(
            dimension_semantics=("arbitrary",)),
    )(block_ex, block_valid, x_sorted, W1, W3, W2)


_TS = 256


def _shared_body(x_ref, w1_ref, w3_ref, w2_ref, g0_ref, g1_ref,
                 cw0_ref, cw1_ref, y_ref):
    x = x_ref[...]
    a1 = lax.dot_general(x, w1_ref[...], (((1,), (1,)), ((), ())),
                         preferred_element_type=jnp.float32)
    a3 = lax.dot_general(x, w3_ref[...], (((1,), (1,)), ((), ())),
                         preferred_element_type=jnp.float32)
    act = a1 * jax.nn.sigmoid(a1) * a3
    y = lax.dot_general(act, w2_ref[...], (((1,), (1,)), ((), ())),
                        preferred_element_type=jnp.float32)
    y_ref[...] = (y + cw0_ref[0, 0, :][:, None] * g0_ref[...]
                  + cw1_ref[0, 0, :][:, None] * g1_ref[...])


def _tc_shared_combine(hf, W1s, W3s, W2s, g0, g1, cw0, cw1):
    return pl.pallas_call(
        _shared_body,
        grid=(T // _TS,),
        in_specs=[
            pl.BlockSpec((_TS, D), lambda i: (i, 0)),
            pl.BlockSpec((H, D), lambda i: (0, 0)),
            pl.BlockSpec((H, D), lambda i: (0, 0)),
            pl.BlockSpec((D, H), lambda i: (0, 0)),
            pl.BlockSpec((_TS, D), lambda i: (i, 0)),
            pl.BlockSpec((_TS, D), lambda i: (i, 0)),
            pl.BlockSpec((1, 1, _TS), lambda i: (i, 0, 0)),
            pl.BlockSpec((1, 1, _TS), lambda i: (i, 0, 0)),
        ],
        out_specs=pl.BlockSpec((_TS, D), lambda i: (i, 0)),
        out_shape=jax.ShapeDtypeStruct((T, D), jnp.float32),
        name="tc_shared_combine",
    )(hf, W1s, W3s, W2s, g0, g1, cw0, cw1)


def _route_body(eb_ref, dst_ref, bex_ref, bval_ref):
    eb = eb_ref[...]
    lane = lax.broadcasted_iota(jnp.int32, (TK, E), 1)
    oh = (eb == lane).astype(jnp.int32)
    c = oh
    k = 1
    while k < TK:
        c = c + jnp.pad(c, ((k, 0), (0, 0)))[:TK]
        k *= 2
    counts = c[TK - 1:TK, :]
    padded = (((counts + TM - 1) // TM) * TM).astype(jnp.float32)
    le = (lax.broadcasted_iota(jnp.int32, (E, E), 0)
          <= lax.broadcasted_iota(jnp.int32, (E, E), 1)).astype(jnp.float32)
    padded_end = lax.dot_general(padded, le, (((1,), (0,)), ((), ())),
                                 preferred_element_type=jnp.float32)
    padded_start = (padded_end - padded).astype(jnp.int32)
    rank = jnp.sum(c * oh, axis=1, keepdims=True) - 1
    startj = jnp.sum(padded_start * oh, axis=1, keepdims=True)
    dst_ref[...] = jnp.broadcast_to(startj + rank, (TK, E))
    bstart = (lax.broadcasted_iota(jnp.int32, (NB, E), 0) * TM
              ).astype(jnp.float32)
    bex = jnp.sum((padded_end <= bstart).astype(jnp.int32),
                  axis=1, keepdims=True)
    bex_ref[...] = jnp.broadcast_to(jnp.minimum(bex, E - 1), (NB, E))
    real_lo = padded_end - padded
    real_hi = real_lo + counts.astype(jnp.float32)
    bval = jnp.sum(((bstart >= real_lo) & (bstart < real_hi)
                    ).astype(jnp.int32), axis=1, keepdims=True)
    bval_ref[...] = jnp.broadcast_to(bval, (NB, E))


def _tc_route_plan(e_bcast):
    return pl.pallas_call(
        _route_body,
        out_shape=(jax.ShapeDtypeStruct((TK, E), jnp.int32),
                   jax.ShapeDtypeStruct((NB, E), jnp.int32),
                   jax.ShapeDtypeStruct((NB, E), jnp.int32)),
        name="tc_route_plan",
    )(e_bcast)


def _route_metadata(ids):
    flat_e = ids.reshape(TK)
    e_bcast = jnp.broadcast_to(flat_e[:, None], (TK, E))
    dst_b, bex_b, bval_b = _tc_route_plan(e_bcast)
    pos = dst_b[:, 0].reshape(T, TOPK)
    return pos, bex_b[:, 0], bval_b[:, 0]


def kernel(h, Wg, W1, W2, W3, W1s, W2s, W3s):
    hf = h.reshape(T, D)
    scores = jax.nn.softmax(hf @ Wg.T, axis=-1)
    v1 = jnp.max(scores, axis=-1)
    a1 = jnp.argmax(scores, axis=-1).astype(jnp.int32)
    masked = jnp.where(
        jax.nn.one_hot(a1, E, dtype=jnp.bool_), -jnp.inf, scores)
    v2 = jnp.max(masked, axis=-1)
    a2 = jnp.argmax(masked, axis=-1).astype(jnp.int32)
    ids = jnp.stack([a1, a2], axis=1)
    vals = jnp.stack([v1, v2], axis=1)
    w = vals / jnp.sum(vals, axis=-1, keepdims=True)

    pos, block_ex, block_valid = _route_metadata(ids)
    pos0 = pos[:, 0].copy()
    pos1 = pos[:, 1].copy()
    cw0 = w[:, 0].reshape(T // _TS, 1, _TS)
    cw1 = w[:, 1].reshape(T // _TS, 1, _TS)

    x_sorted = _make_sc_scatter_rows()(hf, pos0, pos1)
    out_sorted = _tc_grouped_ffn(block_ex, block_valid, x_sorted, W1, W3, W2)
    g0, g1 = _make_sc_gather_outs()(out_sorted, pos0, pos1)
    y = _tc_shared_combine(hf, W1s, W3s, W2s, g0, g1, cw0, cw1)
    return y.reshape(h.shape)
